# baseline (speedup 1.0000x reference)
import functools

import jax
import jax.numpy as jnp
from jax import lax
from jax.experimental import pallas as pl
from jax.experimental.pallas import tpu as pltpu
from jax.experimental.pallas import tpu_sc as plsc

N_NODES = 10000
CH = 128
EMB_CH = 32
NC = 2
NS = 16
NW = NC * NS
CHUNK = 128
NSTEPS = 80
HALF = NSTEPS // 2
PAIRS = NSTEPS // 2
AGG_ROWS = NSTEPS * CHUNK
DUMMY_DST = 10008


def _silu_ln(z, g, b):
    z = z * (1.0 / (1.0 + jnp.exp(-z)))
    mu = jnp.mean(z, axis=-1, keepdims=True)
    var = jnp.mean((z - mu) * (z - mu), axis=-1, keepdims=True)
    return (z - mu) * jax.lax.rsqrt(var + 1e-5) * g + b


def _tc1_body(x_ref, emb_ref, wemb_ref, wx_ref, bin_ref, g_ref, b_ref, h_ref):
    x = x_ref[...]
    blk = x.shape[0]
    cols = lax.broadcasted_iota(jnp.int32, (blk, CH), 1)
    opc = x[:, 0:1].astype(jnp.int32)
    onehot = (cols == opc).astype(jnp.float32)
    w_emb = jnp.dot(emb_ref[...], wemb_ref[...], preferred_element_type=jnp.float32)
    xz = jnp.where(cols == 0, 0.0, x)
    z = (jnp.dot(onehot, w_emb, preferred_element_type=jnp.float32)
         + jnp.dot(xz, wx_ref[...], preferred_element_type=jnp.float32)
         + bin_ref[...])
    h_ref[...] = _silu_ln(z, g_ref[...], b_ref[...])


def _tc2_body(h_ref, agg2_ref, deg2_ref, wmh_ref, wma_ref, bm_ref, g_ref,
              b_ref, wo_ref, bo_ref, out_ref):
    h = h_ref[...]
    a = agg2_ref[0] + agg2_ref[1]
    deg = deg2_ref[0, :, 0:1] + deg2_ref[1, :, 0:1]
    a = a / jnp.maximum(deg, 1.0)
    z = (jnp.dot(h, wmh_ref[...], preferred_element_type=jnp.float32)
         + jnp.dot(a, wma_ref[...], preferred_element_type=jnp.float32)
         + bm_ref[...])
    z = _silu_ln(z, g_ref[...], b_ref[...])
    out_ref[...] = jnp.sum(z * wo_ref[...], axis=-1, keepdims=True) + bo_ref[...]


def _sc_agg_body(h_hbm, srcs_hbm, dsts_hbm, zeros_hbm, agg_out,
                 src_half, dst_half, rows_a, rows_b, agg_sh,
                 sem_ga, sem_gb, sem_sa, sem_sb):
    c = lax.axis_index("c")
    s = lax.axis_index("s")
    wid = s * NC + c
    nchunks = AGG_ROWS // CHUNK

    pltpu.sync_copy(srcs_hbm.at[wid, pl.ds(0, HALF)], src_half)
    pltpu.sync_copy(dsts_hbm.at[wid, pl.ds(0, HALF)], dst_half)
    pltpu.sync_copy(zeros_hbm, rows_a)

    @pl.loop(s, nchunks, step=NS)
    def _zero(j):
        pltpu.sync_copy(rows_a, agg_sh.at[pl.ds(j * CHUNK, CHUNK)])

    plsc.subcore_barrier()

    def _row(j):
        return jnp.where(j >= HALF, j - HALF, j)

    def _gather(j, buf, sem):
        pltpu.async_copy(h_hbm.at[src_half.at[_row(j)]], buf, sem)

    def _wait_gather(j, buf, sem):
        pltpu.make_async_copy(h_hbm.at[src_half.at[_row(j)]], buf, sem).wait()

    def _scat(j, buf, sem):
        pltpu.async_copy(buf, agg_sh.at[dst_half.at[_row(j)]], sem, add=True)

    def _wait_scat(j, buf, sem):
        pltpu.make_async_copy(buf, agg_sh.at[dst_half.at[_row(j)]], sem).wait()

    _gather(0, rows_a, sem_ga)

    @pl.loop(0, PAIRS)
    def _pairs(p):
        j0 = 2 * p
        j1 = j0 + 1

        @pl.when(p > 0)
        def _():
            _wait_scat(j1 - 2, rows_b, sem_sb)

        @pl.when(p == HALF // 2)
        def _():
            pltpu.sync_copy(dsts_hbm.at[wid, pl.ds(HALF, HALF)], dst_half)

        _gather(j1, rows_b, sem_gb)
        _wait_gather(j0, rows_a, sem_ga)
        _scat(j0, rows_a, sem_sa)
        _wait_gather(j1, rows_b, sem_gb)
        _scat(j1, rows_b, sem_sb)

        @pl.when(p == HALF // 2 - 1)
        def _():
            pltpu.sync_copy(srcs_hbm.at[wid, pl.ds(HALF, HALF)], src_half)

        @pl.when(p < PAIRS - 1)
        def _():
            _wait_scat(j0, rows_a, sem_sa)
            _gather(j0 + 2, rows_a, sem_ga)

    _wait_scat(NSTEPS - 2, rows_a, sem_sa)
    _wait_scat(NSTEPS - 1, rows_b, sem_sb)
    plsc.subcore_barrier()

    @pl.loop(s, nchunks, step=NS)
    def _drain(j):
        rows = pl.ds(j * CHUNK, CHUNK)
        pltpu.sync_copy(agg_sh.at[rows], rows_a)
        pltpu.sync_copy(rows_a, agg_out.at[c, rows])


def _sc_deg_body(dsts_hbm, zeros_hbm, ones_hbm, deg_out,
                 dst_all, ones_v, deg_sh, sem):
    c = lax.axis_index("c")
    s = lax.axis_index("s")
    wid = s * NC + c
    nchunks = AGG_ROWS // CHUNK

    pltpu.sync_copy(dsts_hbm.at[wid], dst_all)
    pltpu.sync_copy(zeros_hbm, ones_v)

    @pl.loop(s, nchunks, step=NS)
    def _zero(j):
        pltpu.sync_copy(ones_v, deg_sh.at[pl.ds(j * CHUNK, CHUNK)])

    pltpu.sync_copy(ones_hbm, ones_v)
    plsc.subcore_barrier()

    @pl.loop(0, NSTEPS // 8)
    def _edges(g):
        for k in range(8):
            pltpu.async_copy(ones_v, deg_sh.at[dst_all.at[g * 8 + k]], sem,
                             add=True)
        for k in range(8):
            pltpu.make_async_copy(ones_v, deg_sh.at[dst_all.at[g * 8 + k]],
                                  sem).wait()

    plsc.subcore_barrier()

    @pl.loop(s, nchunks, step=NS)
    def _drain(j):
        rows = pl.ds(j * CHUNK, CHUNK)
        pltpu.sync_copy(deg_sh.at[rows], ones_v)
        pltpu.sync_copy(ones_v, deg_out.at[c, rows])


def kernel(x, edge_index, emb_table, W_in, b_in, ln1_g, ln1_b, W_msg, b_msg,
           ln2_g, ln2_b, W_out, b_out):
    f32 = jnp.float32
    n = x.shape[0]
    e = edge_index.shape[1]

    wemb = W_in[:EMB_CH]
    wx = jnp.concatenate([jnp.zeros((1, CH), f32), W_in[EMB_CH:]], axis=0)
    blk1 = 1000
    h = pl.pallas_call(
        _tc1_body,
        grid=(n // blk1,),
        in_specs=[
            pl.BlockSpec((blk1, CH), lambda i: (i, 0)),
            pl.BlockSpec((CH, EMB_CH), lambda i: (0, 0)),
            pl.BlockSpec((EMB_CH, CH), lambda i: (0, 0)),
            pl.BlockSpec((CH, CH), lambda i: (0, 0)),
            pl.BlockSpec((1, CH), lambda i: (0, 0)),
            pl.BlockSpec((1, CH), lambda i: (0, 0)),
            pl.BlockSpec((1, CH), lambda i: (0, 0)),
        ],
        out_specs=pl.BlockSpec((blk1, CH), lambda i: (i, 0)),
        out_shape=jax.ShapeDtypeStruct((n, CH), f32),
    )(x, emb_table, wemb, wx, b_in.reshape(1, CH), ln1_g.reshape(1, CH),
      ln1_b.reshape(1, CH))

    epw = NSTEPS * CHUNK
    e_pad = epw * NW
    src = edge_index[0].astype(jnp.int32)
    dst = edge_index[1].astype(jnp.int32)
    pad = e_pad - e
    pad_ids = jnp.arange(pad, dtype=jnp.int32)
    srcs = jnp.concatenate([src, pad_ids % jnp.int32(n)])
    dsts = jnp.concatenate(
        [dst, jnp.int32(n) + pad_ids % jnp.int32(AGG_ROWS - n)])
    srcs = srcs.reshape(NW, NSTEPS, CHUNK)
    dsts = dsts.reshape(NW, NSTEPS, CHUNK)
    zeros = jnp.zeros((CHUNK, CH), f32)
    ones = jnp.ones((CHUNK, CH), f32)

    mesh = plsc.VectorSubcoreMesh(core_axis_name="c", subcore_axis_name="s")
    deg2 = pl.kernel(
        _sc_deg_body,
        out_type=jax.ShapeDtypeStruct((NC, AGG_ROWS, CH), f32),
        mesh=mesh,
        scratch_types=[
            pltpu.VMEM((NSTEPS, CHUNK), jnp.int32),
            pltpu.VMEM((CHUNK, CH), f32),
            pltpu.VMEM_SHARED((AGG_ROWS, CH), f32),
            pltpu.SemaphoreType.DMA,
        ],
    )(dsts, zeros, ones)
    agg2 = pl.kernel(
        _sc_agg_body,
        out_type=jax.ShapeDtypeStruct((NC, AGG_ROWS, CH), f32),
        mesh=mesh,
        scratch_types=[
            pltpu.VMEM((HALF, CHUNK), jnp.int32),
            pltpu.VMEM((HALF, CHUNK), jnp.int32),
            pltpu.VMEM((CHUNK, CH), f32),
            pltpu.VMEM((CHUNK, CH), f32),
            pltpu.VMEM_SHARED((AGG_ROWS, CH), f32),
            pltpu.SemaphoreType.DMA,
            pltpu.SemaphoreType.DMA,
            pltpu.SemaphoreType.DMA,
            pltpu.SemaphoreType.DMA,
        ],
    )(h, srcs, dsts, zeros)

    wmh = W_msg[:CH]
    wma = W_msg[CH:]
    blk2 = 1000
    out = pl.pallas_call(
        _tc2_body,
        grid=(n // blk2,),
        in_specs=[
            pl.BlockSpec((blk2, CH), lambda i: (i, 0)),
            pl.BlockSpec((NC, blk2, CH), lambda i: (0, i, 0)),
            pl.BlockSpec((NC, blk2, CH), lambda i: (0, i, 0)),
            pl.BlockSpec((CH, CH), lambda i: (0, 0)),
            pl.BlockSpec((CH, CH), lambda i: (0, 0)),
            pl.BlockSpec((1, CH), lambda i: (0, 0)),
            pl.BlockSpec((1, CH), lambda i: (0, 0)),
            pl.BlockSpec((1, CH), lambda i: (0, 0)),
            pl.BlockSpec((1, CH), lambda i: (0, 0)),
            pl.BlockSpec((1, 1), lambda i: (0, 0)),
        ],
        out_specs=pl.BlockSpec((blk2, 1), lambda i: (i, 0)),
        out_shape=jax.ShapeDtypeStruct((n, 1), f32),
    )(h, agg2, deg2, wmh, wma, b_msg.reshape(1, CH), ln2_g.reshape(1, CH),
      ln2_b.reshape(1, CH), W_out.reshape(1, CH), b_out.reshape(1, 1))
    return out[:, 0]


# speedup vs baseline: 1.0372x; 1.0372x over previous
import functools

import jax
import jax.numpy as jnp
from jax import lax
from jax.experimental import pallas as pl
from jax.experimental.pallas import tpu as pltpu
from jax.experimental.pallas import tpu_sc as plsc

N_NODES = 10000
CH = 128
EMB_CH = 32
NC = 2
NS = 16
NW = NC * NS
CHUNK = 128
NSTEPS = 80
HALF = NSTEPS // 2
PAIRS = NSTEPS // 2
AGG_ROWS = NSTEPS * CHUNK
DUMMY_DST = 10008


def _silu_ln(z, g, b):
    z = z * (1.0 / (1.0 + jnp.exp(-z)))
    mu = jnp.mean(z, axis=-1, keepdims=True)
    var = jnp.mean((z - mu) * (z - mu), axis=-1, keepdims=True)
    return (z - mu) * jax.lax.rsqrt(var + 1e-5) * g + b


def _tc1_body(x_ref, emb_ref, wemb_ref, wx_ref, bin_ref, g_ref, b_ref, h_ref):
    x = x_ref[...]
    blk = x.shape[0]
    cols = lax.broadcasted_iota(jnp.int32, (blk, CH), 1)
    opc = x[:, 0:1].astype(jnp.int32)
    onehot = (cols == opc).astype(jnp.float32)
    w_emb = jnp.dot(emb_ref[...], wemb_ref[...], preferred_element_type=jnp.float32)
    xz = jnp.where(cols == 0, 0.0, x)
    z = (jnp.dot(onehot, w_emb, preferred_element_type=jnp.float32)
         + jnp.dot(xz, wx_ref[...], preferred_element_type=jnp.float32)
         + bin_ref[...])
    h_ref[...] = _silu_ln(z, g_ref[...], b_ref[...])


def _tc2_body(h_ref, agg2_ref, deg_ref, wmh_ref, wma_ref, bm_ref, g_ref,
              b_ref, wo_ref, bo_ref, out_ref):
    h = h_ref[...]
    a = agg2_ref[0] + agg2_ref[1]
    a = a / jnp.maximum(deg_ref[...], 1.0)
    z = (jnp.dot(h, wmh_ref[...], preferred_element_type=jnp.float32)
         + jnp.dot(a, wma_ref[...], preferred_element_type=jnp.float32)
         + bm_ref[...])
    z = _silu_ln(z, g_ref[...], b_ref[...])
    out_ref[...] = jnp.sum(z * wo_ref[...], axis=-1, keepdims=True) + bo_ref[...]


def _sc_agg_body(h_hbm, srcs_hbm, dsts_hbm, zeros_hbm, agg_out,
                 src_half, dst_half, rows_a, rows_b, agg_sh,
                 sem_ga, sem_gb, sem_sa, sem_sb):
    c = lax.axis_index("c")
    s = lax.axis_index("s")
    wid = s * NC + c
    nchunks = AGG_ROWS // CHUNK

    pltpu.sync_copy(srcs_hbm.at[wid, pl.ds(0, HALF)], src_half)
    pltpu.sync_copy(dsts_hbm.at[wid, pl.ds(0, HALF)], dst_half)
    pltpu.sync_copy(zeros_hbm, rows_a)

    @pl.loop(s, nchunks, step=NS)
    def _zero(j):
        pltpu.sync_copy(rows_a, agg_sh.at[pl.ds(j * CHUNK, CHUNK)])

    plsc.subcore_barrier()

    def _row(j):
        return jnp.where(j >= HALF, j - HALF, j)

    def _gather(j, buf, sem):
        pltpu.async_copy(h_hbm.at[src_half.at[_row(j)]], buf, sem)

    def _wait_gather(j, buf, sem):
        pltpu.make_async_copy(h_hbm.at[src_half.at[_row(j)]], buf, sem).wait()

    def _scat(j, buf, sem):
        pltpu.async_copy(buf, agg_sh.at[dst_half.at[_row(j)]], sem, add=True)

    def _wait_scat(j, buf, sem):
        pltpu.make_async_copy(buf, agg_sh.at[dst_half.at[_row(j)]], sem).wait()

    _gather(0, rows_a, sem_ga)

    @pl.loop(0, PAIRS)
    def _pairs(p):
        j0 = 2 * p
        j1 = j0 + 1

        @pl.when(p > 0)
        def _():
            _wait_scat(j1 - 2, rows_b, sem_sb)

        @pl.when(p == HALF // 2)
        def _():
            pltpu.sync_copy(dsts_hbm.at[wid, pl.ds(HALF, HALF)], dst_half)

        _gather(j1, rows_b, sem_gb)
        _wait_gather(j0, rows_a, sem_ga)
        _scat(j0, rows_a, sem_sa)
        _wait_gather(j1, rows_b, sem_gb)
        _scat(j1, rows_b, sem_sb)

        @pl.when(p == HALF // 2 - 1)
        def _():
            pltpu.sync_copy(srcs_hbm.at[wid, pl.ds(HALF, HALF)], src_half)

        @pl.when(p < PAIRS - 1)
        def _():
            _wait_scat(j0, rows_a, sem_sa)
            _gather(j0 + 2, rows_a, sem_ga)

    _wait_scat(NSTEPS - 2, rows_a, sem_sa)
    _wait_scat(NSTEPS - 1, rows_b, sem_sb)
    plsc.subcore_barrier()

    @pl.loop(s, nchunks, step=NS)
    def _drain(j):
        rows = pl.ds(j * CHUNK, CHUNK)
        pltpu.sync_copy(agg_sh.at[rows], rows_a)
        pltpu.sync_copy(rows_a, agg_out.at[c, rows])


def _sc_deg_body(dsts_hbm, zeros_hbm, ones_hbm, deg_out,
                 dst_all, ones_v, deg_sh, sem):
    c = lax.axis_index("c")
    s = lax.axis_index("s")
    wid = s * NC + c
    nchunks = AGG_ROWS // CHUNK

    pltpu.sync_copy(dsts_hbm.at[wid], dst_all)
    pltpu.sync_copy(zeros_hbm, ones_v)

    @pl.loop(s, nchunks, step=NS)
    def _zero(j):
        pltpu.sync_copy(ones_v, deg_sh.at[pl.ds(j * CHUNK, CHUNK)])

    pltpu.sync_copy(ones_hbm, ones_v)
    plsc.subcore_barrier()

    @pl.loop(0, NSTEPS // 8)
    def _edges(g):
        for k in range(8):
            pltpu.async_copy(ones_v, deg_sh.at[dst_all.at[g * 8 + k]], sem,
                             add=True)
        for k in range(8):
            pltpu.make_async_copy(ones_v, deg_sh.at[dst_all.at[g * 8 + k]],
                                  sem).wait()

    plsc.subcore_barrier()

    @pl.loop(s, nchunks, step=NS)
    def _drain(j):
        rows = pl.ds(j * CHUNK, CHUNK)
        pltpu.sync_copy(deg_sh.at[rows], ones_v)
        pltpu.sync_copy(ones_v, deg_out.at[c, rows])


def kernel(x, edge_index, emb_table, W_in, b_in, ln1_g, ln1_b, W_msg, b_msg,
           ln2_g, ln2_b, W_out, b_out):
    f32 = jnp.float32
    n = x.shape[0]
    e = edge_index.shape[1]

    wemb = W_in[:EMB_CH]
    wx = jnp.concatenate([jnp.zeros((1, CH), f32), W_in[EMB_CH:]], axis=0)
    blk1 = 1000
    h = pl.pallas_call(
        _tc1_body,
        grid=(n // blk1,),
        in_specs=[
            pl.BlockSpec((blk1, CH), lambda i: (i, 0)),
            pl.BlockSpec((CH, EMB_CH), lambda i: (0, 0)),
            pl.BlockSpec((EMB_CH, CH), lambda i: (0, 0)),
            pl.BlockSpec((CH, CH), lambda i: (0, 0)),
            pl.BlockSpec((1, CH), lambda i: (0, 0)),
            pl.BlockSpec((1, CH), lambda i: (0, 0)),
            pl.BlockSpec((1, CH), lambda i: (0, 0)),
        ],
        out_specs=pl.BlockSpec((blk1, CH), lambda i: (i, 0)),
        out_shape=jax.ShapeDtypeStruct((n, CH), f32),
    )(x, emb_table, wemb, wx, b_in.reshape(1, CH), ln1_g.reshape(1, CH),
      ln1_b.reshape(1, CH))

    epw = NSTEPS * CHUNK
    e_pad = epw * NW
    src = edge_index[0].astype(jnp.int32)
    dst = edge_index[1].astype(jnp.int32)
    pad = e_pad - e
    pad_ids = jnp.arange(pad, dtype=jnp.int32)
    srcs = jnp.concatenate([src, pad_ids % jnp.int32(n)])
    dsts = jnp.concatenate(
        [dst, jnp.int32(n) + pad_ids % jnp.int32(AGG_ROWS - n)])
    srcs = srcs.reshape(NW, NSTEPS, CHUNK)
    dsts = dsts.reshape(NW, NSTEPS, CHUNK)
    zeros = jnp.zeros((CHUNK, CH), f32)
    ones = jnp.ones((CHUNK, CH), f32)

    mesh = plsc.VectorSubcoreMesh(core_axis_name="c", subcore_axis_name="s")
    deg2 = pl.kernel(
        _sc_deg_body,
        out_type=jax.ShapeDtypeStruct((NC, AGG_ROWS, CH), f32),
        mesh=mesh,
        scratch_types=[
            pltpu.VMEM((NSTEPS, CHUNK), jnp.int32),
            pltpu.VMEM((CHUNK, CH), f32),
            pltpu.VMEM_SHARED((AGG_ROWS, CH), f32),
            pltpu.SemaphoreType.DMA,
        ],
    )(dsts, zeros, ones)
    agg2 = pl.kernel(
        _sc_agg_body,
        out_type=jax.ShapeDtypeStruct((NC, AGG_ROWS, CH), f32),
        mesh=mesh,
        scratch_types=[
            pltpu.VMEM((HALF, CHUNK), jnp.int32),
            pltpu.VMEM((HALF, CHUNK), jnp.int32),
            pltpu.VMEM((CHUNK, CH), f32),
            pltpu.VMEM((CHUNK, CH), f32),
            pltpu.VMEM_SHARED((AGG_ROWS, CH), f32),
            pltpu.SemaphoreType.DMA,
            pltpu.SemaphoreType.DMA,
            pltpu.SemaphoreType.DMA,
            pltpu.SemaphoreType.DMA,
        ],
    )(h, srcs, dsts, zeros)
    deg_col = (deg2[0, :, 0] + deg2[1, :, 0])[:n].reshape(n, 1)

    wmh = W_msg[:CH]
    wma = W_msg[CH:]
    blk2 = 1000
    out = pl.pallas_call(
        _tc2_body,
        grid=(n // blk2,),
        in_specs=[
            pl.BlockSpec((blk2, CH), lambda i: (i, 0)),
            pl.BlockSpec((NC, blk2, CH), lambda i: (0, i, 0)),
            pl.BlockSpec((blk2, 1), lambda i: (i, 0)),
            pl.BlockSpec((CH, CH), lambda i: (0, 0)),
            pl.BlockSpec((CH, CH), lambda i: (0, 0)),
            pl.BlockSpec((1, CH), lambda i: (0, 0)),
            pl.BlockSpec((1, CH), lambda i: (0, 0)),
            pl.BlockSpec((1, CH), lambda i: (0, 0)),
            pl.BlockSpec((1, CH), lambda i: (0, 0)),
            pl.BlockSpec((1, 1), lambda i: (0, 0)),
        ],
        out_specs=pl.BlockSpec((blk2, 1), lambda i: (i, 0)),
        out_shape=jax.ShapeDtypeStruct((n, 1), f32),
    )(h, agg2, deg_col, wmh, wma, b_msg.reshape(1, CH), ln2_g.reshape(1, CH),
      ln2_b.reshape(1, CH), W_out.reshape(1, CH), b_out.reshape(1, 1))
    return out[:, 0]


# speedup vs baseline: 1.3108x; 1.2638x over previous
import functools

import jax
import jax.numpy as jnp
from jax import lax
from jax.experimental import pallas as pl
from jax.experimental.pallas import tpu as pltpu
from jax.experimental.pallas import tpu_sc as plsc

N_NODES = 10000
CH = 128
EMB_CH = 32
NC = 2
NS = 16
NW = NC * NS
CHUNK = 128
NSTEPS = 80
HALF = NSTEPS // 2
PAIRS = NSTEPS // 2
AGG_ROWS = NSTEPS * CHUNK
DUMMY_DST = 10008


def _silu_ln(z, g, b):
    z = z * (1.0 / (1.0 + jnp.exp(-z)))
    mu = jnp.mean(z, axis=-1, keepdims=True)
    var = jnp.mean((z - mu) * (z - mu), axis=-1, keepdims=True)
    return (z - mu) * jax.lax.rsqrt(var + 1e-5) * g + b


def _tc1_body(x_ref, emb_ref, wemb_ref, wx_ref, bin_ref, g_ref, b_ref, h_ref):
    x = x_ref[...]
    blk = x.shape[0]
    cols = lax.broadcasted_iota(jnp.int32, (blk, CH), 1)
    opc = x[:, 0:1].astype(jnp.int32)
    onehot = (cols == opc).astype(jnp.float32)
    w_emb = jnp.dot(emb_ref[...], wemb_ref[...], preferred_element_type=jnp.float32)
    xz = jnp.where(cols == 0, 0.0, x)
    z = (jnp.dot(onehot, w_emb, preferred_element_type=jnp.float32)
         + jnp.dot(xz, wx_ref[...], preferred_element_type=jnp.float32)
         + bin_ref[...])
    h_ref[...] = _silu_ln(z, g_ref[...], b_ref[...])


def _tc2_body(h_ref, agg2_ref, deg_ref, wmh_ref, wma_ref, bm_ref, g_ref,
              b_ref, wo_ref, bo_ref, out_ref):
    h = h_ref[...]
    a = agg2_ref[0] + agg2_ref[1]
    a = a / jnp.maximum(deg_ref[...], 1.0)
    z = (jnp.dot(h, wmh_ref[...], preferred_element_type=jnp.float32)
         + jnp.dot(a, wma_ref[...], preferred_element_type=jnp.float32)
         + bm_ref[...])
    z = _silu_ln(z, g_ref[...], b_ref[...])
    out_ref[...] = jnp.sum(z * wo_ref[...], axis=-1, keepdims=True) + bo_ref[...]


def _sc_agg_body(h_hbm, srcs_hbm, dsts_hbm, zeros_hbm, agg_out,
                 src_half, dst_half, rows_a, rows_b, agg_sh,
                 sem_ga, sem_gb, sem_sa, sem_sb):
    c = lax.axis_index("c")
    s = lax.axis_index("s")
    wid = s * NC + c
    nchunks = AGG_ROWS // CHUNK

    pltpu.sync_copy(srcs_hbm.at[wid, pl.ds(0, HALF)], src_half)
    pltpu.sync_copy(dsts_hbm.at[wid, pl.ds(0, HALF)], dst_half)
    pltpu.sync_copy(zeros_hbm, rows_a)

    @pl.loop(s, nchunks, step=NS)
    def _zero(j):
        pltpu.sync_copy(rows_a, agg_sh.at[pl.ds(j * CHUNK, CHUNK)])

    plsc.subcore_barrier()

    def _row(j):
        return jnp.where(j >= HALF, j - HALF, j)

    def _gather(j, buf, sem):
        pltpu.async_copy(h_hbm.at[src_half.at[_row(j)]], buf, sem)

    def _wait_gather(j, buf, sem):
        pltpu.make_async_copy(h_hbm.at[src_half.at[_row(j)]], buf, sem).wait()

    def _scat(j, buf, sem):
        pltpu.async_copy(buf, agg_sh.at[dst_half.at[_row(j)]], sem, add=True)

    def _wait_scat(j, buf, sem):
        pltpu.make_async_copy(buf, agg_sh.at[dst_half.at[_row(j)]], sem).wait()

    _gather(0, rows_a, sem_ga)

    @pl.loop(0, PAIRS)
    def _pairs(p):
        j0 = 2 * p
        j1 = j0 + 1

        @pl.when(p > 0)
        def _():
            _wait_scat(j1 - 2, rows_b, sem_sb)

        @pl.when(p == HALF // 2)
        def _():
            pltpu.sync_copy(dsts_hbm.at[wid, pl.ds(HALF, HALF)], dst_half)

        _gather(j1, rows_b, sem_gb)
        _wait_gather(j0, rows_a, sem_ga)
        _scat(j0, rows_a, sem_sa)
        _wait_gather(j1, rows_b, sem_gb)
        _scat(j1, rows_b, sem_sb)

        @pl.when(p == HALF // 2 - 1)
        def _():
            pltpu.sync_copy(srcs_hbm.at[wid, pl.ds(HALF, HALF)], src_half)

        @pl.when(p < PAIRS - 1)
        def _():
            _wait_scat(j0, rows_a, sem_sa)
            _gather(j0 + 2, rows_a, sem_ga)

    _wait_scat(NSTEPS - 2, rows_a, sem_sa)
    _wait_scat(NSTEPS - 1, rows_b, sem_sb)
    plsc.subcore_barrier()

    @pl.loop(s, nchunks, step=NS)
    def _drain(j):
        rows = pl.ds(j * CHUNK, CHUNK)
        pltpu.sync_copy(agg_sh.at[rows], rows_a)
        pltpu.sync_copy(rows_a, agg_out.at[c, rows])


def _sc_deg_body(dsts_hbm, zeros_hbm, ones_hbm, deg_out,
                 dst_all, ones_v, deg_sh, sem):
    c = lax.axis_index("c")
    s = lax.axis_index("s")
    wid = s * NC + c
    nchunks = AGG_ROWS // CHUNK

    pltpu.sync_copy(dsts_hbm.at[wid], dst_all)
    pltpu.sync_copy(zeros_hbm, ones_v)

    @pl.loop(s, nchunks, step=NS)
    def _zero(j):
        pltpu.sync_copy(ones_v, deg_sh.at[pl.ds(j * CHUNK, CHUNK)])

    pltpu.sync_copy(ones_hbm, ones_v)
    plsc.subcore_barrier()

    @pl.loop(0, NSTEPS // 8)
    def _edges(g):
        for k in range(8):
            pltpu.async_copy(ones_v, deg_sh.at[dst_all.at[g * 8 + k]], sem,
                             add=True)
        for k in range(8):
            pltpu.make_async_copy(ones_v, deg_sh.at[dst_all.at[g * 8 + k]],
                                  sem).wait()

    plsc.subcore_barrier()

    @pl.loop(s, nchunks, step=NS)
    def _drain(j):
        rows = pl.ds(j * CHUNK, CHUNK)
        pltpu.sync_copy(deg_sh.at[rows], ones_v)
        pltpu.sync_copy(ones_v, deg_out.at[c, rows])


def kernel(x, edge_index, emb_table, W_in, b_in, ln1_g, ln1_b, W_msg, b_msg,
           ln2_g, ln2_b, W_out, b_out):
    f32 = jnp.float32
    n = x.shape[0]
    e = edge_index.shape[1]

    wemb = W_in[:EMB_CH]
    wx = jnp.concatenate([jnp.zeros((1, CH), f32), W_in[EMB_CH:]], axis=0)
    blk1 = 1000
    h = pl.pallas_call(
        _tc1_body,
        grid=(n // blk1,),
        in_specs=[
            pl.BlockSpec((blk1, CH), lambda i: (i, 0)),
            pl.BlockSpec((CH, EMB_CH), lambda i: (0, 0)),
            pl.BlockSpec((EMB_CH, CH), lambda i: (0, 0)),
            pl.BlockSpec((CH, CH), lambda i: (0, 0)),
            pl.BlockSpec((1, CH), lambda i: (0, 0)),
            pl.BlockSpec((1, CH), lambda i: (0, 0)),
            pl.BlockSpec((1, CH), lambda i: (0, 0)),
        ],
        out_specs=pl.BlockSpec((blk1, CH), lambda i: (i, 0)),
        out_shape=jax.ShapeDtypeStruct((n, CH), f32),
    )(x, emb_table, wemb, wx, b_in.reshape(1, CH), ln1_g.reshape(1, CH),
      ln1_b.reshape(1, CH))

    epw = NSTEPS * CHUNK
    e_pad = epw * NW
    src = edge_index[0].astype(jnp.int32)
    dst = edge_index[1].astype(jnp.int32)
    pad = e_pad - e
    pad_ids = jnp.arange(pad, dtype=jnp.int32)
    srcs = jnp.concatenate([src, pad_ids % jnp.int32(n)])
    dsts = jnp.concatenate(
        [dst, jnp.int32(n) + pad_ids % jnp.int32(AGG_ROWS - n)])
    srcs = srcs.reshape(NW, NSTEPS, CHUNK)
    dsts = dsts.reshape(NW, NSTEPS, CHUNK)
    zeros = jnp.zeros((CHUNK, CH), f32)
    zeros16 = jnp.zeros((CHUNK, 16), f32)
    ones16 = jnp.ones((CHUNK, 16), f32)

    mesh = plsc.VectorSubcoreMesh(core_axis_name="c", subcore_axis_name="s")
    deg2 = pl.kernel(
        _sc_deg_body,
        out_type=jax.ShapeDtypeStruct((NC, AGG_ROWS, 16), f32),
        mesh=mesh,
        compiler_params=pltpu.CompilerParams(use_tc_tiling_on_sc=False),
        scratch_types=[
            pltpu.VMEM((NSTEPS, CHUNK), jnp.int32),
            pltpu.VMEM((CHUNK, 16), f32),
            pltpu.VMEM_SHARED((AGG_ROWS, 16), f32),
            pltpu.SemaphoreType.DMA,
        ],
    )(dsts, zeros16, ones16)
    agg2 = pl.kernel(
        _sc_agg_body,
        out_type=jax.ShapeDtypeStruct((NC, AGG_ROWS, CH), f32),
        mesh=mesh,
        scratch_types=[
            pltpu.VMEM((HALF, CHUNK), jnp.int32),
            pltpu.VMEM((HALF, CHUNK), jnp.int32),
            pltpu.VMEM((CHUNK, CH), f32),
            pltpu.VMEM((CHUNK, CH), f32),
            pltpu.VMEM_SHARED((AGG_ROWS, CH), f32),
            pltpu.SemaphoreType.DMA,
            pltpu.SemaphoreType.DMA,
            pltpu.SemaphoreType.DMA,
            pltpu.SemaphoreType.DMA,
        ],
    )(h, srcs, dsts, zeros)
    deg_col = (deg2[0, :, 0] + deg2[1, :, 0])[:n].reshape(n, 1)

    wmh = W_msg[:CH]
    wma = W_msg[CH:]
    blk2 = 1000
    out = pl.pallas_call(
        _tc2_body,
        grid=(n // blk2,),
        in_specs=[
            pl.BlockSpec((blk2, CH), lambda i: (i, 0)),
            pl.BlockSpec((NC, blk2, CH), lambda i: (0, i, 0)),
            pl.BlockSpec((blk2, 1), lambda i: (i, 0)),
            pl.BlockSpec((CH, CH), lambda i: (0, 0)),
            pl.BlockSpec((CH, CH), lambda i: (0, 0)),
            pl.BlockSpec((1, CH), lambda i: (0, 0)),
            pl.BlockSpec((1, CH), lambda i: (0, 0)),
            pl.BlockSpec((1, CH), lambda i: (0, 0)),
            pl.BlockSpec((1, CH), lambda i: (0, 0)),
            pl.BlockSpec((1, 1), lambda i: (0, 0)),
        ],
        out_specs=pl.BlockSpec((blk2, 1), lambda i: (i, 0)),
        out_shape=jax.ShapeDtypeStruct((n, 1), f32),
    )(h, agg2, deg_col, wmh, wma, b_msg.reshape(1, CH), ln2_g.reshape(1, CH),
      ln2_b.reshape(1, CH), W_out.reshape(1, CH), b_out.reshape(1, 1))
    return out[:, 0]


# speedup vs baseline: 1.3145x; 1.0028x over previous
import functools

import jax
import jax.numpy as jnp
from jax import lax
from jax.experimental import pallas as pl
from jax.experimental.pallas import tpu as pltpu
from jax.experimental.pallas import tpu_sc as plsc

N_NODES = 10000
CH = 128
EMB_CH = 32
NC = 2
NS = 16
NW = NC * NS
CHUNK = 128
NSTEPS = 80
HALF = NSTEPS // 2
PAIRS = NSTEPS // 2
AGG_ROWS = NSTEPS * CHUNK
DUMMY_DST = 10008


def _silu_ln(z, g, b):
    z = z * (1.0 / (1.0 + jnp.exp(-z)))
    mu = jnp.mean(z, axis=-1, keepdims=True)
    var = jnp.mean((z - mu) * (z - mu), axis=-1, keepdims=True)
    return (z - mu) * jax.lax.rsqrt(var + 1e-5) * g + b


def _tc1_body(x_ref, emb_ref, wemb_ref, wx_ref, bin_ref, g_ref, b_ref, h_ref):
    x = x_ref[...]
    blk = x.shape[0]
    cols = lax.broadcasted_iota(jnp.int32, (blk, CH), 1)
    opc = x[:, 0:1].astype(jnp.int32)
    onehot = (cols == opc).astype(jnp.float32)
    w_emb = jnp.dot(emb_ref[...], wemb_ref[...], preferred_element_type=jnp.float32)
    xz = jnp.where(cols == 0, 0.0, x)
    z = (jnp.dot(onehot, w_emb, preferred_element_type=jnp.float32)
         + jnp.dot(xz, wx_ref[...], preferred_element_type=jnp.float32)
         + bin_ref[...])
    h_ref[...] = _silu_ln(z, g_ref[...], b_ref[...])


def _tc2a_body(h_ref, wmh_ref, bm_ref, zh_ref):
    zh_ref[...] = (jnp.dot(h_ref[...], wmh_ref[...],
                           preferred_element_type=jnp.float32) + bm_ref[...])


def _tc2b_body(zh_ref, agg2_ref, deg_ref, wma_ref, g_ref,
               b_ref, wo_ref, bo_ref, out_ref):
    a = agg2_ref[0] + agg2_ref[1]
    a = a / jnp.maximum(deg_ref[...], 1.0)
    z = zh_ref[...] + jnp.dot(a, wma_ref[...], preferred_element_type=jnp.float32)
    z = _silu_ln(z, g_ref[...], b_ref[...])
    out_ref[...] = jnp.sum(z * wo_ref[...], axis=-1, keepdims=True) + bo_ref[...]


def _sc_agg_body(h_hbm, srcs_hbm, dsts_hbm, zeros_hbm, agg_out,
                 src_half, dst_half, rows_a, rows_b, agg_sh,
                 sem_ga, sem_gb, sem_sa, sem_sb):
    c = lax.axis_index("c")
    s = lax.axis_index("s")
    wid = s * NC + c
    nchunks = AGG_ROWS // CHUNK

    pltpu.sync_copy(srcs_hbm.at[wid, pl.ds(0, HALF)], src_half)
    pltpu.sync_copy(dsts_hbm.at[wid, pl.ds(0, HALF)], dst_half)
    pltpu.sync_copy(zeros_hbm, rows_a)

    @pl.loop(s, nchunks, step=NS)
    def _zero(j):
        pltpu.sync_copy(rows_a, agg_sh.at[pl.ds(j * CHUNK, CHUNK)])

    plsc.subcore_barrier()

    def _row(j):
        return jnp.where(j >= HALF, j - HALF, j)

    def _gather(j, buf, sem):
        pltpu.async_copy(h_hbm.at[src_half.at[_row(j)]], buf, sem)

    def _wait_gather(j, buf, sem):
        pltpu.make_async_copy(h_hbm.at[src_half.at[_row(j)]], buf, sem).wait()

    def _scat(j, buf, sem):
        pltpu.async_copy(buf, agg_sh.at[dst_half.at[_row(j)]], sem, add=True)

    def _wait_scat(j, buf, sem):
        pltpu.make_async_copy(buf, agg_sh.at[dst_half.at[_row(j)]], sem).wait()

    _gather(0, rows_a, sem_ga)

    @pl.loop(0, PAIRS)
    def _pairs(p):
        j0 = 2 * p
        j1 = j0 + 1

        @pl.when(p > 0)
        def _():
            _wait_scat(j1 - 2, rows_b, sem_sb)

        @pl.when(p == HALF // 2)
        def _():
            pltpu.sync_copy(dsts_hbm.at[wid, pl.ds(HALF, HALF)], dst_half)

        _gather(j1, rows_b, sem_gb)
        _wait_gather(j0, rows_a, sem_ga)
        _scat(j0, rows_a, sem_sa)
        _wait_gather(j1, rows_b, sem_gb)
        _scat(j1, rows_b, sem_sb)

        @pl.when(p == HALF // 2 - 1)
        def _():
            pltpu.sync_copy(srcs_hbm.at[wid, pl.ds(HALF, HALF)], src_half)

        @pl.when(p < PAIRS - 1)
        def _():
            _wait_scat(j0, rows_a, sem_sa)
            _gather(j0 + 2, rows_a, sem_ga)

    _wait_scat(NSTEPS - 2, rows_a, sem_sa)
    _wait_scat(NSTEPS - 1, rows_b, sem_sb)
    plsc.subcore_barrier()

    @pl.loop(s, nchunks, step=NS)
    def _drain(j):
        rows = pl.ds(j * CHUNK, CHUNK)
        pltpu.sync_copy(agg_sh.at[rows], rows_a)
        pltpu.sync_copy(rows_a, agg_out.at[c, rows])


def _sc_deg_body(dsts_hbm, zeros_hbm, ones_hbm, deg_out,
                 dst_all, ones_v, deg_sh, sem):
    c = lax.axis_index("c")
    s = lax.axis_index("s")
    wid = s * NC + c
    nchunks = AGG_ROWS // CHUNK

    pltpu.sync_copy(dsts_hbm.at[wid], dst_all)
    pltpu.sync_copy(zeros_hbm, ones_v)

    @pl.loop(s, nchunks, step=NS)
    def _zero(j):
        pltpu.sync_copy(ones_v, deg_sh.at[pl.ds(j * CHUNK, CHUNK)])

    pltpu.sync_copy(ones_hbm, ones_v)
    plsc.subcore_barrier()

    @pl.loop(0, NSTEPS // 8)
    def _edges(g):
        for k in range(8):
            pltpu.async_copy(ones_v, deg_sh.at[dst_all.at[g * 8 + k]], sem,
                             add=True)
        for k in range(8):
            pltpu.make_async_copy(ones_v, deg_sh.at[dst_all.at[g * 8 + k]],
                                  sem).wait()

    plsc.subcore_barrier()

    @pl.loop(s, nchunks, step=NS)
    def _drain(j):
        rows = pl.ds(j * CHUNK, CHUNK)
        pltpu.sync_copy(deg_sh.at[rows], ones_v)
        pltpu.sync_copy(ones_v, deg_out.at[c, rows])


def kernel(x, edge_index, emb_table, W_in, b_in, ln1_g, ln1_b, W_msg, b_msg,
           ln2_g, ln2_b, W_out, b_out):
    f32 = jnp.float32
    n = x.shape[0]
    e = edge_index.shape[1]

    wemb = W_in[:EMB_CH]
    wx = jnp.concatenate([jnp.zeros((1, CH), f32), W_in[EMB_CH:]], axis=0)
    blk1 = 1000
    h = pl.pallas_call(
        _tc1_body,
        grid=(n // blk1,),
        in_specs=[
            pl.BlockSpec((blk1, CH), lambda i: (i, 0)),
            pl.BlockSpec((CH, EMB_CH), lambda i: (0, 0)),
            pl.BlockSpec((EMB_CH, CH), lambda i: (0, 0)),
            pl.BlockSpec((CH, CH), lambda i: (0, 0)),
            pl.BlockSpec((1, CH), lambda i: (0, 0)),
            pl.BlockSpec((1, CH), lambda i: (0, 0)),
            pl.BlockSpec((1, CH), lambda i: (0, 0)),
        ],
        out_specs=pl.BlockSpec((blk1, CH), lambda i: (i, 0)),
        out_shape=jax.ShapeDtypeStruct((n, CH), f32),
    )(x, emb_table, wemb, wx, b_in.reshape(1, CH), ln1_g.reshape(1, CH),
      ln1_b.reshape(1, CH))

    epw = NSTEPS * CHUNK
    e_pad = epw * NW
    src = edge_index[0].astype(jnp.int32)
    dst = edge_index[1].astype(jnp.int32)
    pad = e_pad - e
    pad_ids = jnp.arange(pad, dtype=jnp.int32)
    srcs = jnp.concatenate([src, pad_ids % jnp.int32(n)])
    dsts = jnp.concatenate(
        [dst, jnp.int32(n) + pad_ids % jnp.int32(AGG_ROWS - n)])
    srcs = srcs.reshape(NW, NSTEPS, CHUNK)
    dsts = dsts.reshape(NW, NSTEPS, CHUNK)
    zeros = jnp.zeros((CHUNK, CH), f32)
    zeros16 = jnp.zeros((CHUNK, 16), f32)
    ones16 = jnp.ones((CHUNK, 16), f32)

    mesh = plsc.VectorSubcoreMesh(core_axis_name="c", subcore_axis_name="s")
    deg2 = pl.kernel(
        _sc_deg_body,
        out_type=jax.ShapeDtypeStruct((NC, AGG_ROWS, 16), f32),
        mesh=mesh,
        compiler_params=pltpu.CompilerParams(use_tc_tiling_on_sc=False),
        scratch_types=[
            pltpu.VMEM((NSTEPS, CHUNK), jnp.int32),
            pltpu.VMEM((CHUNK, 16), f32),
            pltpu.VMEM_SHARED((AGG_ROWS, 16), f32),
            pltpu.SemaphoreType.DMA,
        ],
    )(dsts, zeros16, ones16)
    agg2 = pl.kernel(
        _sc_agg_body,
        out_type=jax.ShapeDtypeStruct((NC, AGG_ROWS, CH), f32),
        mesh=mesh,
        scratch_types=[
            pltpu.VMEM((HALF, CHUNK), jnp.int32),
            pltpu.VMEM((HALF, CHUNK), jnp.int32),
            pltpu.VMEM((CHUNK, CH), f32),
            pltpu.VMEM((CHUNK, CH), f32),
            pltpu.VMEM_SHARED((AGG_ROWS, CH), f32),
            pltpu.SemaphoreType.DMA,
            pltpu.SemaphoreType.DMA,
            pltpu.SemaphoreType.DMA,
            pltpu.SemaphoreType.DMA,
        ],
    )(h, srcs, dsts, zeros)
    deg_col = (deg2[0, :, 0] + deg2[1, :, 0])[:n].reshape(n, 1)

    wmh = W_msg[:CH]
    wma = W_msg[CH:]
    blk2 = 1000
    zh = pl.pallas_call(
        _tc2a_body,
        grid=(n // blk2,),
        in_specs=[
            pl.BlockSpec((blk2, CH), lambda i: (i, 0)),
            pl.BlockSpec((CH, CH), lambda i: (0, 0)),
            pl.BlockSpec((1, CH), lambda i: (0, 0)),
        ],
        out_specs=pl.BlockSpec((blk2, CH), lambda i: (i, 0)),
        out_shape=jax.ShapeDtypeStruct((n, CH), f32),
    )(h, wmh, b_msg.reshape(1, CH))
    out = pl.pallas_call(
        _tc2b_body,
        grid=(n // blk2,),
        in_specs=[
            pl.BlockSpec((blk2, CH), lambda i: (i, 0)),
            pl.BlockSpec((NC, blk2, CH), lambda i: (0, i, 0)),
            pl.BlockSpec((blk2, 1), lambda i: (i, 0)),
            pl.BlockSpec((CH, CH), lambda i: (0, 0)),
            pl.BlockSpec((1, CH), lambda i: (0, 0)),
            pl.BlockSpec((1, CH), lambda i: (0, 0)),
            pl.BlockSpec((1, CH), lambda i: (0, 0)),
            pl.BlockSpec((1, 1), lambda i: (0, 0)),
        ],
        out_specs=pl.BlockSpec((blk2, 1), lambda i: (i, 0)),
        out_shape=jax.ShapeDtypeStruct((n, 1), f32),
    )(zh, agg2, deg_col, wma, ln2_g.reshape(1, CH),
      ln2_b.reshape(1, CH), W_out.reshape(1, CH), b_out.reshape(1, 1))
    return out[:, 0]


# speedup vs baseline: 1.3358x; 1.0162x over previous
import functools

import jax
import jax.numpy as jnp
from jax import lax
from jax.experimental import pallas as pl
from jax.experimental.pallas import tpu as pltpu
from jax.experimental.pallas import tpu_sc as plsc

N_NODES = 10000
CH = 128
EMB_CH = 32
NC = 2
NS = 16
NW = NC * NS
CHUNK = 128
NSTEPS = 80
HALF = NSTEPS // 2
PAIRS = NSTEPS // 2
AGG_ROWS = NSTEPS * CHUNK
DUMMY_DST = 10008


def _silu_ln(z, g, b):
    z = z * (1.0 / (1.0 + jnp.exp(-z)))
    mu = jnp.mean(z, axis=-1, keepdims=True)
    var = jnp.mean((z - mu) * (z - mu), axis=-1, keepdims=True)
    return (z - mu) * jax.lax.rsqrt(var + 1e-5) * g + b


def _tc1_body(x_ref, emb_ref, wemb_ref, wx_ref, bin_ref, g_ref, b_ref, h_ref):
    x = x_ref[...]
    blk = x.shape[0]
    cols = lax.broadcasted_iota(jnp.int32, (blk, CH), 1)
    opc = x[:, 0:1].astype(jnp.int32)
    onehot = (cols == opc).astype(jnp.float32)
    w_emb = jnp.dot(emb_ref[...], wemb_ref[...], preferred_element_type=jnp.float32)
    xz = jnp.where(cols == 0, 0.0, x)
    z = (jnp.dot(onehot, w_emb, preferred_element_type=jnp.float32)
         + jnp.dot(xz, wx_ref[...], preferred_element_type=jnp.float32)
         + bin_ref[...])
    h_ref[...] = _silu_ln(z, g_ref[...], b_ref[...])


def _tc2a_body(h_ref, wmh_ref, bm_ref, zh_ref):
    zh_ref[...] = (jnp.dot(h_ref[...], wmh_ref[...],
                           preferred_element_type=jnp.float32) + bm_ref[...])


def _tc2b_body(zh_ref, agg2_ref, deg2_ref, wma_ref, g_ref,
               b_ref, wo_ref, bo_ref, out_ref):
    a = agg2_ref[0] + agg2_ref[1]
    deg = deg2_ref[0, :, 0:1] + deg2_ref[1, :, 0:1]
    a = a / jnp.maximum(deg, 1.0)
    z = zh_ref[...] + jnp.dot(a, wma_ref[...], preferred_element_type=jnp.float32)
    z = _silu_ln(z, g_ref[...], b_ref[...])
    out_ref[...] = jnp.sum(z * wo_ref[...], axis=-1, keepdims=True) + bo_ref[...]


def _sc_agg_body(h_hbm, srcs_hbm, dsts_hbm, zeros_hbm, agg_out,
                 src_half, dst_half, rows_a, rows_b, agg_sh,
                 sem_ga, sem_gb, sem_sa, sem_sb):
    c = lax.axis_index("c")
    s = lax.axis_index("s")
    wid = s * NC + c
    nchunks = AGG_ROWS // CHUNK

    pltpu.sync_copy(srcs_hbm.at[wid, pl.ds(0, HALF)], src_half)
    pltpu.sync_copy(dsts_hbm.at[wid, pl.ds(0, HALF)], dst_half)
    pltpu.sync_copy(zeros_hbm, rows_a)

    @pl.loop(s, nchunks, step=NS)
    def _zero(j):
        pltpu.sync_copy(rows_a, agg_sh.at[pl.ds(j * CHUNK, CHUNK)])

    plsc.subcore_barrier()

    def _row(j):
        return jnp.where(j >= HALF, j - HALF, j)

    def _gather(j, buf, sem):
        pltpu.async_copy(h_hbm.at[src_half.at[_row(j)]], buf, sem)

    def _wait_gather(j, buf, sem):
        pltpu.make_async_copy(h_hbm.at[src_half.at[_row(j)]], buf, sem).wait()

    def _scat(j, buf, sem):
        pltpu.async_copy(buf, agg_sh.at[dst_half.at[_row(j)]], sem, add=True)

    def _wait_scat(j, buf, sem):
        pltpu.make_async_copy(buf, agg_sh.at[dst_half.at[_row(j)]], sem).wait()

    _gather(0, rows_a, sem_ga)

    @pl.loop(0, PAIRS)
    def _pairs(p):
        j0 = 2 * p
        j1 = j0 + 1

        @pl.when(p > 0)
        def _():
            _wait_scat(j1 - 2, rows_b, sem_sb)

        @pl.when(p == HALF // 2)
        def _():
            pltpu.sync_copy(dsts_hbm.at[wid, pl.ds(HALF, HALF)], dst_half)

        _gather(j1, rows_b, sem_gb)
        _wait_gather(j0, rows_a, sem_ga)
        _scat(j0, rows_a, sem_sa)
        _wait_gather(j1, rows_b, sem_gb)
        _scat(j1, rows_b, sem_sb)

        @pl.when(p == HALF // 2 - 1)
        def _():
            pltpu.sync_copy(srcs_hbm.at[wid, pl.ds(HALF, HALF)], src_half)

        @pl.when(p < PAIRS - 1)
        def _():
            _wait_scat(j0, rows_a, sem_sa)
            _gather(j0 + 2, rows_a, sem_ga)

    _wait_scat(NSTEPS - 2, rows_a, sem_sa)
    _wait_scat(NSTEPS - 1, rows_b, sem_sb)
    plsc.subcore_barrier()

    @pl.loop(s, nchunks, step=NS)
    def _drain(j):
        rows = pl.ds(j * CHUNK, CHUNK)
        pltpu.sync_copy(agg_sh.at[rows], rows_a)
        pltpu.sync_copy(rows_a, agg_out.at[c, rows])


def _sc_deg_body(dsts_hbm, zeros_hbm, ones_hbm, deg_out,
                 dst_all, ones_v, deg_sh, sem):
    c = lax.axis_index("c")
    s = lax.axis_index("s")
    wid = s * NC + c
    nchunks = AGG_ROWS // CHUNK

    pltpu.sync_copy(dsts_hbm.at[wid], dst_all)
    pltpu.sync_copy(zeros_hbm, ones_v)

    @pl.loop(s, nchunks, step=NS)
    def _zero(j):
        pltpu.sync_copy(ones_v, deg_sh.at[pl.ds(j * CHUNK, CHUNK)])

    pltpu.sync_copy(ones_hbm, ones_v)
    plsc.subcore_barrier()

    @pl.loop(0, NSTEPS // 8)
    def _edges(g):
        for k in range(8):
            pltpu.async_copy(ones_v, deg_sh.at[dst_all.at[g * 8 + k]], sem,
                             add=True)
        for k in range(8):
            pltpu.make_async_copy(ones_v, deg_sh.at[dst_all.at[g * 8 + k]],
                                  sem).wait()

    plsc.subcore_barrier()

    @pl.loop(s, nchunks, step=NS)
    def _drain(j):
        rows = pl.ds(j * CHUNK, CHUNK)
        pltpu.sync_copy(deg_sh.at[rows], ones_v)
        pltpu.sync_copy(ones_v, deg_out.at[c, rows])


def kernel(x, edge_index, emb_table, W_in, b_in, ln1_g, ln1_b, W_msg, b_msg,
           ln2_g, ln2_b, W_out, b_out):
    f32 = jnp.float32
    n = x.shape[0]
    e = edge_index.shape[1]

    wemb = W_in[:EMB_CH]
    wx = jnp.concatenate([jnp.zeros((1, CH), f32), W_in[EMB_CH:]], axis=0)
    blk1 = 2000
    h = pl.pallas_call(
        _tc1_body,
        grid=(n // blk1,),
        in_specs=[
            pl.BlockSpec((blk1, CH), lambda i: (i, 0)),
            pl.BlockSpec((CH, EMB_CH), lambda i: (0, 0)),
            pl.BlockSpec((EMB_CH, CH), lambda i: (0, 0)),
            pl.BlockSpec((CH, CH), lambda i: (0, 0)),
            pl.BlockSpec((1, CH), lambda i: (0, 0)),
            pl.BlockSpec((1, CH), lambda i: (0, 0)),
            pl.BlockSpec((1, CH), lambda i: (0, 0)),
        ],
        out_specs=pl.BlockSpec((blk1, CH), lambda i: (i, 0)),
        out_shape=jax.ShapeDtypeStruct((n, CH), f32),
    )(x, emb_table, wemb, wx, b_in.reshape(1, CH), ln1_g.reshape(1, CH),
      ln1_b.reshape(1, CH))

    epw = NSTEPS * CHUNK
    e_pad = epw * NW
    src = edge_index[0].astype(jnp.int32)
    dst = edge_index[1].astype(jnp.int32)
    pad = e_pad - e
    pad_ids = jnp.arange(pad, dtype=jnp.int32)
    srcs = jnp.concatenate([src, pad_ids % jnp.int32(n)])
    dsts = jnp.concatenate(
        [dst, jnp.int32(n) + pad_ids % jnp.int32(AGG_ROWS - n)])
    srcs = srcs.reshape(NW, NSTEPS, CHUNK)
    dsts = dsts.reshape(NW, NSTEPS, CHUNK)
    zeros = jnp.zeros((CHUNK, CH), f32)
    zeros16 = jnp.zeros((CHUNK, 16), f32)
    ones16 = jnp.ones((CHUNK, 16), f32)

    mesh = plsc.VectorSubcoreMesh(core_axis_name="c", subcore_axis_name="s")
    deg2 = pl.kernel(
        _sc_deg_body,
        out_type=jax.ShapeDtypeStruct((NC, AGG_ROWS, 16), f32),
        mesh=mesh,
        compiler_params=pltpu.CompilerParams(use_tc_tiling_on_sc=False),
        scratch_types=[
            pltpu.VMEM((NSTEPS, CHUNK), jnp.int32),
            pltpu.VMEM((CHUNK, 16), f32),
            pltpu.VMEM_SHARED((AGG_ROWS, 16), f32),
            pltpu.SemaphoreType.DMA,
        ],
    )(dsts, zeros16, ones16)
    agg2 = pl.kernel(
        _sc_agg_body,
        out_type=jax.ShapeDtypeStruct((NC, AGG_ROWS, CH), f32),
        mesh=mesh,
        scratch_types=[
            pltpu.VMEM((HALF, CHUNK), jnp.int32),
            pltpu.VMEM((HALF, CHUNK), jnp.int32),
            pltpu.VMEM((CHUNK, CH), f32),
            pltpu.VMEM((CHUNK, CH), f32),
            pltpu.VMEM_SHARED((AGG_ROWS, CH), f32),
            pltpu.SemaphoreType.DMA,
            pltpu.SemaphoreType.DMA,
            pltpu.SemaphoreType.DMA,
            pltpu.SemaphoreType.DMA,
        ],
    )(h, srcs, dsts, zeros)

    wmh = W_msg[:CH]
    wma = W_msg[CH:]
    blk2 = 2000
    zh = pl.pallas_call(
        _tc2a_body,
        grid=(n // blk2,),
        in_specs=[
            pl.BlockSpec((blk2, CH), lambda i: (i, 0)),
            pl.BlockSpec((CH, CH), lambda i: (0, 0)),
            pl.BlockSpec((1, CH), lambda i: (0, 0)),
        ],
        out_specs=pl.BlockSpec((blk2, CH), lambda i: (i, 0)),
        out_shape=jax.ShapeDtypeStruct((n, CH), f32),
    )(h, wmh, b_msg.reshape(1, CH))
    out = pl.pallas_call(
        _tc2b_body,
        grid=(n // blk2,),
        in_specs=[
            pl.BlockSpec((blk2, CH), lambda i: (i, 0)),
            pl.BlockSpec((NC, blk2, CH), lambda i: (0, i, 0)),
            pl.BlockSpec((NC, blk2, 16), lambda i: (0, i, 0)),
            pl.BlockSpec((CH, CH), lambda i: (0, 0)),
            pl.BlockSpec((1, CH), lambda i: (0, 0)),
            pl.BlockSpec((1, CH), lambda i: (0, 0)),
            pl.BlockSpec((1, CH), lambda i: (0, 0)),
            pl.BlockSpec((1, 1), lambda i: (0, 0)),
        ],
        out_specs=pl.BlockSpec((blk2, 1), lambda i: (i, 0)),
        out_shape=jax.ShapeDtypeStruct((n, 1), f32),
    )(zh, agg2, deg2, wma, ln2_g.reshape(1, CH),
      ln2_b.reshape(1, CH), W_out.reshape(1, CH), b_out.reshape(1, 1))
    return out[:, 0]
